# baseline (speedup 1.0000x reference)
import functools

import jax
import jax.numpy as jnp
from jax import lax
from jax.experimental import pallas as pl
from jax.experimental.pallas import tpu as pltpu
from jax.experimental.pallas import tpu_sc as plsc

_VOCAB = 1000000
_EMBED = 64
_BATCH = 16384
_NC = 2
_NS = 16
_NW = _NC * _NS
_BPW = _BATCH // _NW
_LANES = 16
_BW = 256
_STRIPE = 31232
_LAST_LO = 31 * _STRIPE
_TAIL_LO = 999936
_LCAP = 4096
_WCAP = 256


def _stream_body(target_hbm, context_hbm, wt_in, wt_out, aux_in, aux_out,
                 gt_hbm, gc_hbm,
                 idx_v, lst_v, wk_v, blk0_v, blk1_v, rows_v, shr_v,
                 sem, semA, semB, semTA, semTB):
    sid = lax.axis_index("s")
    wid = lax.axis_index("s") * _NC + lax.axis_index("c")
    lo = jnp.where(wid == 31, _LAST_LO, wid * _STRIPE)
    width = jnp.where(wid == 31, _VOCAB - _LAST_LO, _STRIPE)
    nblk = jnp.where(wid == 31, 125, 122)
    lanes = lax.iota(jnp.int32, _LANES)

    def one_table(idx_hbm, wt, aux, g_hbm):
        pltpu.sync_copy(idx_hbm, idx_v)

        def filt(k, off):
            v = idx_v[pl.ds(k * _LANES, _LANES)]
            lv = v - lo
            m = (lv >= 0) & (lv < width)
            packed = (lv << 14) | (k * _LANES + lanes)
            off_c = jnp.minimum(off, _LCAP - _LANES)
            plsc.store_compressed(lst_v.at[pl.ds(off_c, _LANES)], packed,
                                  mask=m)
            return off + plsc.all_reduce_population_count(m)[0]

        nhits = lax.fori_loop(0, _BATCH // _LANES, filt, 0)
        nhits = jnp.minimum(nhits, _LCAP - _LANES)
        nvec = (nhits + _LANES - 1) // _LANES

        def fire_hbm(j, par, bsem):
            @pl.when(j < 124)
            def _():
                src0 = pl.multiple_of(lo + j * _BW, 128)
                for k in range(_EMBED // 8):
                    pltpu.async_copy(
                        wt.at[k, :, pl.ds(src0, _BW)],
                        shr_v.at[sid, par, pl.ds(k * 8, 8)], bsem)

            @pl.when(j >= 124)
            def _():
                pltpu.async_copy(aux, shr_v.at[sid, par, :, pl.ds(0, 128)],
                                 bsem)

        def wait_hbm(j, par, bsem):
            @pl.when(j < 124)
            def _():
                for k in range(_EMBED // 8):
                    pltpu.make_async_copy(
                        wt.at[0, :, pl.ds(0, _BW)],
                        shr_v.at[sid, par, pl.ds(k * 8, 8)], bsem).wait()

            @pl.when(j >= 124)
            def _():
                pltpu.make_async_copy(
                    aux, shr_v.at[sid, par, :, pl.ds(0, 128)], bsem).wait()

        def fire_ts(par, buf, tsem):
            pltpu.async_copy(shr_v.at[sid, par], buf, tsem)

        def wait_ts(par, buf, tsem):
            pltpu.make_async_copy(shr_v.at[sid, par], buf, tsem).wait()

        def process(blk, blk_v):
            blo = blk * _BW
            bhi = blo + _BW

            def scan_vec(k, wcnt):
                pk = lst_v[pl.ds(pl.multiple_of(k * _LANES, 8), _LANES)]
                lv = pk >> 14
                valid = (k * _LANES + lanes) < nhits
                hitm = (lv >= blo) & (lv < bhi) & valid
                wc = jnp.minimum(wcnt, _WCAP - _LANES)
                plsc.store_compressed(wk_v.at[pl.ds(wc, _LANES)], pk,
                                      mask=hitm)
                return wcnt + plsc.all_reduce_population_count(hitm)[0]

            wcnt = lax.fori_loop(0, nvec, scan_vec, 0)
            wcnt = jnp.minimum(wcnt, _WCAP - _LANES)

            def extract(h, c):
                pkv = plsc.load_gather(wk_v, [jnp.broadcast_to(h, (_LANES,))])
                p = pkv[0]
                col = (p >> 14) - blo
                b = p & 16383
                slot = h

                for cc in range(_EMBED // _LANES):
                    g = plsc.load_gather(
                        blk_v,
                        [lanes + cc * _LANES,
                         jnp.broadcast_to(col, (_LANES,))])
                    rows_v[pl.ds(slot * _EMBED + cc * _LANES, _LANES)] = g
                pltpu.async_copy(
                    rows_v.at[pl.ds(slot * _EMBED, _EMBED)],
                    g_hbm.at[pl.ds(b * _EMBED, _EMBED)], sem)
                return c

            lax.fori_loop(0, wcnt, extract, 0)

            def drain(h, c):
                pltpu.make_async_copy(
                    g_hbm.at[pl.ds(0, _EMBED)],
                    rows_v.at[pl.ds(0, _EMBED)], sem).wait()
                return c

            lax.fori_loop(0, wcnt, drain, 0)

        fire_hbm(jnp.int32(0), 0, semA)

        def pipe_body(i, carry):
            @pl.when((i & 1) == 0)
            def _():
                @pl.when(i < nblk)
                def _():
                    wait_hbm(i, 0, semA)
                    fire_ts(0, blk0_v, semTA)

                @pl.when(i >= 1)
                def _():
                    wait_ts(1, blk1_v, semTB)

                @pl.when(i + 1 < nblk)
                def _():
                    fire_hbm(i + 1, 1, semB)

                @pl.when(i >= 1)
                def _():
                    process(i - 1, blk1_v)

            @pl.when((i & 1) == 1)
            def _():
                @pl.when(i < nblk)
                def _():
                    wait_hbm(i, 1, semB)
                    fire_ts(1, blk1_v, semTB)

                @pl.when(i >= 1)
                def _():
                    wait_ts(0, blk0_v, semTA)

                @pl.when(i + 1 < nblk)
                def _():
                    fire_hbm(i + 1, 0, semA)

                @pl.when(i >= 1)
                def _():
                    process(i - 1, blk0_v)

            return carry

        lax.fori_loop(0, nblk + 1, pipe_body, 0)

    one_table(target_hbm, wt_in, aux_in, gt_hbm)
    one_table(context_hbm, wt_out, aux_out, gc_hbm)


def _dot_body(gt_hbm, gc_hbm, out_hbm, rows_a, rows_c, out_v, sem):
    wid = lax.axis_index("s") * _NC + lax.axis_index("c")
    base = wid * _BPW
    pltpu.sync_copy(gt_hbm.at[pl.ds(base * _EMBED, _BPW * _EMBED)], rows_a)
    pltpu.sync_copy(gc_hbm.at[pl.ds(base * _EMBED, _BPW * _EMBED)], rows_c)
    lanes = lax.iota(jnp.int32, _LANES)

    def group_body(g, carry):
        row0 = g * _LANES
        out_vec = jnp.zeros((_LANES,), jnp.float32)
        for i in range(_LANES):
            off = (row0 + i) * _EMBED
            p = jnp.zeros((_LANES,), jnp.float32)
            for c in range(_EMBED // _LANES):
                a = rows_a[pl.ds(off + c * _LANES, _LANES)]
                b = rows_c[pl.ds(off + c * _LANES, _LANES)]
                p = p + a * b
            s = jnp.sum(p)
            out_vec = jnp.where(lanes == i, s, out_vec)
        out_v[pl.ds(row0, _LANES)] = out_vec
        return carry

    lax.fori_loop(0, _BPW // _LANES, group_body, 0)
    pltpu.sync_copy(out_v, out_hbm.at[pl.ds(base, _BPW)])


def kernel(target, context, W_in, W_out):
    mesh = plsc.VectorSubcoreMesh(core_axis_name="c", subcore_axis_name="s")
    params = pltpu.CompilerParams(needs_layout_passes=False)

    stream = functools.partial(
        pl.kernel,
        out_type=(jax.ShapeDtypeStruct((_BATCH * _EMBED,), jnp.float32),
                  jax.ShapeDtypeStruct((_BATCH * _EMBED,), jnp.float32)),
        mesh=mesh,
        compiler_params=params,
        scratch_types=[
            pltpu.VMEM((_BATCH,), jnp.int32),
            pltpu.VMEM((_LCAP,), jnp.int32),
            pltpu.VMEM((_WCAP,), jnp.int32),
            pltpu.VMEM((_EMBED, _BW), jnp.float32),
            pltpu.VMEM((_EMBED, _BW), jnp.float32),
            pltpu.VMEM((_WCAP * _EMBED,), jnp.float32),
            pltpu.VMEM_SHARED((_NS, 2, _EMBED, _BW), jnp.float32),
            pltpu.SemaphoreType.DMA,
            pltpu.SemaphoreType.DMA,
            pltpu.SemaphoreType.DMA,
            pltpu.SemaphoreType.DMA,
            pltpu.SemaphoreType.DMA,
        ],
    )(_stream_body)

    dot = functools.partial(
        pl.kernel,
        out_type=jax.ShapeDtypeStruct((_BATCH,), jnp.float32),
        mesh=mesh,
        compiler_params=params,
        scratch_types=[
            pltpu.VMEM((_BPW * _EMBED,), jnp.float32),
            pltpu.VMEM((_BPW * _EMBED,), jnp.float32),
            pltpu.VMEM((_BPW,), jnp.float32),
            pltpu.SemaphoreType.DMA,
        ],
    )(_dot_body)

    aux_in = jnp.pad(W_in[_TAIL_LO:].T, ((0, 0), (0, 64)))
    aux_out = jnp.pad(W_out[_TAIL_LO:].T, ((0, 0), (0, 64)))
    gt, gc = stream(target.astype(jnp.int32), context.astype(jnp.int32),
                    W_in.T.reshape(8, 8, _VOCAB),
                    W_out.T.reshape(8, 8, _VOCAB),
                    aux_in, aux_out)
    return dot(gt, gc)


# speedup vs baseline: 1.3126x; 1.3126x over previous
import functools

import jax
import jax.numpy as jnp
from jax import lax
from jax.experimental import pallas as pl
from jax.experimental.pallas import tpu as pltpu
from jax.experimental.pallas import tpu_sc as plsc

_VOCAB = 1000000
_EMBED = 64
_BATCH = 16384
_NC = 2
_NS = 16
_NW = _NC * _NS
_BPW = _BATCH // _NW
_LANES = 16
_BW = 512
_STRIPE = 31232
_LAST_LO = 31 * _STRIPE
_TAIL_LO = 999936
_LCAP = 4096
_WCAP = 256


def _stream_body(target_hbm, context_hbm, wt_in, wt_out, aux_in, aux_out,
                 gt_hbm, gc_hbm,
                 idx_v, lst_v, wk_v, blk0_v, blk1_v, rows_v,
                 sem, semA, semB):
    wid = lax.axis_index("s") * _NC + lax.axis_index("c")
    lo = jnp.where(wid == 31, _LAST_LO, wid * _STRIPE)
    width = jnp.where(wid == 31, _VOCAB - _LAST_LO, _STRIPE)
    nblk = jnp.where(wid == 31, 63, 61)
    lanes = lax.iota(jnp.int32, _LANES)

    def one_table(idx_hbm, wt, aux, g_hbm):
        pltpu.sync_copy(idx_hbm, idx_v)

        def filt(k, off):
            v = idx_v[pl.ds(k * _LANES, _LANES)]
            lv = v - lo
            m = (lv >= 0) & (lv < width)
            packed = (lv << 14) | (k * _LANES + lanes)
            off_c = jnp.minimum(off, _LCAP - _LANES)
            plsc.store_compressed(lst_v.at[pl.ds(off_c, _LANES)], packed,
                                  mask=m)
            return off + plsc.all_reduce_population_count(m)[0]

        nhits = lax.fori_loop(0, _BATCH // _LANES, filt, 0)
        nhits = jnp.minimum(nhits, _LCAP - _LANES)
        nvec = (nhits + _LANES - 1) // _LANES

        def fire(j, buf, bsem):
            @pl.when(j < 62)
            def _():
                src0 = pl.multiple_of(lo + j * _BW, 128)
                for k in range(_EMBED // 8):
                    pltpu.async_copy(wt.at[k, :, pl.ds(src0, _BW)],
                                     buf.at[pl.ds(k * 8, 8)], bsem)

            @pl.when(j >= 62)
            def _():
                pltpu.async_copy(aux, buf.at[:, pl.ds(0, 128)], bsem)

        def wait_blk(j, buf, bsem):
            @pl.when(j < 62)
            def _():
                for k in range(_EMBED // 8):
                    pltpu.make_async_copy(
                        wt.at[0, :, pl.ds(0, _BW)],
                        buf.at[pl.ds(k * 8, 8)], bsem).wait()

            @pl.when(j >= 62)
            def _():
                pltpu.make_async_copy(
                    aux, buf.at[:, pl.ds(0, 128)], bsem).wait()

        def process(blk, blk_v):
            blo = blk * _BW
            bhi = blo + _BW

            def scan_vec(k, wcnt):
                pk = lst_v[pl.ds(pl.multiple_of(k * _LANES, 8), _LANES)]
                lv = pk >> 14
                valid = (k * _LANES + lanes) < nhits
                hitm = (lv >= blo) & (lv < bhi) & valid
                wc = jnp.minimum(wcnt, _WCAP - _LANES)
                plsc.store_compressed(wk_v.at[pl.ds(wc, _LANES)], pk,
                                      mask=hitm)
                return wcnt + plsc.all_reduce_population_count(hitm)[0]

            wcnt = lax.fori_loop(0, nvec, scan_vec, 0)
            wcnt = jnp.minimum(wcnt, _WCAP - _LANES)

            def extract(h, c):
                pkv = plsc.load_gather(wk_v, [jnp.broadcast_to(h, (_LANES,))])
                p = pkv[0]
                col = (p >> 14) - blo
                b = p & 16383
                slot = h

                for cc in range(_EMBED // _LANES):
                    g = plsc.load_gather(
                        blk_v,
                        [lanes + cc * _LANES,
                         jnp.broadcast_to(col, (_LANES,))])
                    rows_v[pl.ds(slot * _EMBED + cc * _LANES, _LANES)] = g
                pltpu.async_copy(
                    rows_v.at[pl.ds(slot * _EMBED, _EMBED)],
                    g_hbm.at[pl.ds(b * _EMBED, _EMBED)], sem)
                return c

            lax.fori_loop(0, wcnt, extract, 0)

            def drain(h, c):
                pltpu.make_async_copy(
                    g_hbm.at[pl.ds(0, _EMBED)],
                    rows_v.at[pl.ds(0, _EMBED)], sem).wait()
                return c

            lax.fori_loop(0, wcnt, drain, 0)

        fire(jnp.int32(0), blk0_v, semA)

        def block_body(blk, carry):
            @pl.when((blk & 1) == 0)
            def _():
                @pl.when(blk + 1 < nblk)
                def _():
                    fire(blk + 1, blk1_v, semB)
                wait_blk(blk, blk0_v, semA)
                process(blk, blk0_v)

            @pl.when((blk & 1) == 1)
            def _():
                @pl.when(blk + 1 < nblk)
                def _():
                    fire(blk + 1, blk0_v, semA)
                wait_blk(blk, blk1_v, semB)
                process(blk, blk1_v)

            return carry

        lax.fori_loop(0, nblk, block_body, 0)

    one_table(target_hbm, wt_in, aux_in, gt_hbm)
    one_table(context_hbm, wt_out, aux_out, gc_hbm)


def _dot_body(gt_hbm, gc_hbm, out_hbm, rows_a, rows_c, out_v, sem):
    wid = lax.axis_index("s") * _NC + lax.axis_index("c")
    base = wid * _BPW
    pltpu.sync_copy(gt_hbm.at[pl.ds(base * _EMBED, _BPW * _EMBED)], rows_a)
    pltpu.sync_copy(gc_hbm.at[pl.ds(base * _EMBED, _BPW * _EMBED)], rows_c)
    lanes = lax.iota(jnp.int32, _LANES)

    def group_body(g, carry):
        row0 = g * _LANES
        out_vec = jnp.zeros((_LANES,), jnp.float32)
        for i in range(_LANES):
            off = (row0 + i) * _EMBED
            p = jnp.zeros((_LANES,), jnp.float32)
            for c in range(_EMBED // _LANES):
                a = rows_a[pl.ds(off + c * _LANES, _LANES)]
                b = rows_c[pl.ds(off + c * _LANES, _LANES)]
                p = p + a * b
            s = jnp.sum(p)
            out_vec = jnp.where(lanes == i, s, out_vec)
        out_v[pl.ds(row0, _LANES)] = out_vec
        return carry

    lax.fori_loop(0, _BPW // _LANES, group_body, 0)
    pltpu.sync_copy(out_v, out_hbm.at[pl.ds(base, _BPW)])


def kernel(target, context, W_in, W_out):
    mesh = plsc.VectorSubcoreMesh(core_axis_name="c", subcore_axis_name="s")
    params = pltpu.CompilerParams(needs_layout_passes=False)

    stream = functools.partial(
        pl.kernel,
        out_type=(jax.ShapeDtypeStruct((_BATCH * _EMBED,), jnp.float32),
                  jax.ShapeDtypeStruct((_BATCH * _EMBED,), jnp.float32)),
        mesh=mesh,
        compiler_params=params,
        scratch_types=[
            pltpu.VMEM((_BATCH,), jnp.int32),
            pltpu.VMEM((_LCAP,), jnp.int32),
            pltpu.VMEM((_WCAP,), jnp.int32),
            pltpu.VMEM((_EMBED, _BW), jnp.float32),
            pltpu.VMEM((_EMBED, _BW), jnp.float32),
            pltpu.VMEM((_WCAP * _EMBED,), jnp.float32),
            pltpu.SemaphoreType.DMA,
            pltpu.SemaphoreType.DMA,
            pltpu.SemaphoreType.DMA,
        ],
    )(_stream_body)

    dot = functools.partial(
        pl.kernel,
        out_type=jax.ShapeDtypeStruct((_BATCH,), jnp.float32),
        mesh=mesh,
        compiler_params=params,
        scratch_types=[
            pltpu.VMEM((_BPW * _EMBED,), jnp.float32),
            pltpu.VMEM((_BPW * _EMBED,), jnp.float32),
            pltpu.VMEM((_BPW,), jnp.float32),
            pltpu.SemaphoreType.DMA,
        ],
    )(_dot_body)

    aux_in = jnp.pad(W_in[_TAIL_LO:].T, ((0, 0), (0, 64)))
    aux_out = jnp.pad(W_out[_TAIL_LO:].T, ((0, 0), (0, 64)))
    gt, gc = stream(target.astype(jnp.int32), context.astype(jnp.int32),
                    W_in.T.reshape(8, 8, _VOCAB),
                    W_out.T.reshape(8, 8, _VOCAB),
                    aux_in, aux_out)
    return dot(gt, gc)
